# baseline (speedup 1.0000x reference)
import functools

import jax
import jax.numpy as jnp
from jax import lax
from jax.experimental import pallas as pl
from jax.experimental.pallas import tpu as pltpu
from jax.experimental.pallas import tpu_sc as plsc

_NC = 2
_NS = 16
_LN = 16
_NW = _NC * _NS


def _mesh():
    return plsc.VectorSubcoreMesh(core_axis_name="c", subcore_axis_name="s")


def _p0a_build(E, RN):
    SEG = RN // _NW
    SEGP = ((SEG + 15) // 16) * 16
    CB = 2000

    @functools.partial(
        pl.kernel,
        out_type=jax.ShapeDtypeStruct((RN,), jnp.float32),
        mesh=_mesh(),
        compiler_params=pltpu.CompilerParams(needs_layout_passes=False),
        scratch_types=[
            pltpu.VMEM((SEGP,), jnp.float32),
            pltpu.VMEM((CB,), jnp.int32),
        ],
    )
    def p0a(comb_hbm, invtab_hbm, degv, cbuf):
        c = lax.axis_index("c")
        s = lax.axis_index("s")
        w = c * _NS + s
        lo = w * SEG
        zero = jnp.zeros((_LN,), jnp.float32)
        ones = jnp.ones((_LN,), jnp.float32)

        def zbody(i, _):
            degv[pl.ds(i * _LN, _LN)] = zero
            return 0
        lax.fori_loop(0, SEGP // _LN, zbody, 0)

        def chunk(j, _):
            pltpu.sync_copy(comb_hbm.at[pl.ds(j * CB, CB)], cbuf)

            def inner(k, _):
                cv = cbuf[pl.ds(k * _LN, _LN)]
                lv = cv - lo
                m = (lv >= 0) & (lv < SEG)
                lvc = jnp.clip(lv, 0, SEGP - 1)
                plsc.addupdate_scatter(degv, [lvc], ones, mask=m)
                return 0
            lax.fori_loop(0, CB // _LN, inner, 0)
            return 0
        lax.fori_loop(0, E // CB, chunk, 0)

        def ibody(i, _):
            dv = degv[pl.ds(i * _LN, _LN)]
            degv[pl.ds(i * _LN, _LN)] = 1.0 / jnp.maximum(dv, 1.0)
            return 0
        lax.fori_loop(0, SEGP // _LN, ibody, 0)

        pltpu.sync_copy(degv.at[pl.ds(0, SEG)], invtab_hbm.at[pl.ds(lo, SEG)])

    return p0a


def _p0b_build(E, RN):
    CW = 40
    NCH = E // (_NW * CW)

    @functools.partial(
        pl.kernel,
        out_type=jax.ShapeDtypeStruct((E,), jnp.float32),
        mesh=_mesh(),
        compiler_params=pltpu.CompilerParams(needs_layout_passes=False),
        scratch_types=[
            pltpu.VMEM((NCH, CW), jnp.int32),
            pltpu.VMEM((CW,), jnp.float32),
            pltpu.VMEM((CW,), jnp.float32),
            pltpu.SemaphoreType.DMA,
            pltpu.SemaphoreType.DMA,
        ],
    )
    def p0b(comb3_hbm, invtab_hbm, inve_hbm, cidx, ib0, ib1, s0, s1):
        c = lax.axis_index("c")
        s = lax.axis_index("s")
        w = c * _NS + s
        base = w * NCH
        pltpu.sync_copy(comb3_hbm.at[w], cidx)

        pltpu.async_copy(invtab_hbm.at[cidx.at[0]], ib0, s0)
        pltpu.async_copy(invtab_hbm.at[cidx.at[1]], ib1, s1)

        def step(j, ib, sem):
            pltpu.make_async_copy(invtab_hbm.at[cidx.at[j]], ib, sem).wait()
            pltpu.sync_copy(ib, inve_hbm.at[pl.ds((base + j) * CW, CW)])

        def pair(i, _):
            j0 = i * 2
            step(j0, ib0, s0)

            @pl.when(j0 + 2 < NCH)
            def _():
                pltpu.async_copy(invtab_hbm.at[cidx.at[j0 + 2]], ib0, s0)

            step(j0 + 1, ib1, s1)

            @pl.when(j0 + 3 < NCH)
            def _():
                pltpu.async_copy(invtab_hbm.at[cidx.at[j0 + 3]], ib1, s1)
            return 0
        lax.fori_loop(0, NCH // 2, pair, 0)
        if NCH % 2:
            step(NCH - 1, ib0, s0)

    return p0b


def _s2_build(NH, E, HD):
    C = 80
    NCH = E // (_NS * C)
    NROW = NH // _NS
    ZR = 64

    @functools.partial(
        pl.kernel,
        out_type=jax.ShapeDtypeStruct((_NC, NH, HD), jnp.float32),
        mesh=_mesh(),
        compiler_params=pltpu.CompilerParams(needs_layout_passes=False),
        scratch_types=[
            pltpu.VMEM((NCH, C), jnp.int32),
            pltpu.VMEM((NCH, C), jnp.int32),
            pltpu.VMEM((NCH, C), jnp.float32),
            pltpu.VMEM((C, HD), jnp.float32),
            pltpu.VMEM((C, HD), jnp.float32),
            pltpu.VMEM((ZR, HD), jnp.float32),
            pltpu.VMEM_SHARED((NH, HD), jnp.float32),
            pltpu.SemaphoreType.DMA,
            pltpu.SemaphoreType.DMA,
        ],
    )
    def s2(h_hbm, gidx_hbm, dst_hbm, inv_hbm, out_hbm,
           gidxv, dstv, invv, rb0, rb1, zbuf, acc, gs0, gs1):
        c = lax.axis_index("c")
        s = lax.axis_index("s")
        pltpu.sync_copy(gidx_hbm.at[c, s], gidxv)
        pltpu.sync_copy(dst_hbm.at[s], dstv)
        pltpu.sync_copy(inv_hbm.at[s], invv)

        zero = jnp.zeros((_LN,), jnp.float32)

        def zrow(i, _):
            def zcol(g, _):
                zbuf[i, pl.ds(g * _LN, _LN)] = zero
                return 0
            lax.fori_loop(0, HD // _LN, zcol, 0)
            return 0
        lax.fori_loop(0, ZR, zrow, 0)
        for k in range(NROW // ZR):
            pltpu.sync_copy(zbuf, acc.at[pl.ds(s * NROW + k * ZR, ZR)])
        plsc.subcore_barrier()

        pltpu.async_copy(h_hbm.at[gidxv.at[0]], rb0, gs0)
        pltpu.async_copy(h_hbm.at[gidxv.at[1]], rb1, gs1)
        iota = lax.iota(jnp.int32, _LN)

        def process(j, rb, gs):
            pltpu.make_async_copy(h_hbm.at[gidxv.at[j]], rb, gs).wait()

            def grp(g, _):
                sc = invv[j, pl.ds(g * _LN, _LN)]
                rows16 = iota + g * _LN
                for col in range(HD):
                    colv = jnp.full((_LN,), col, jnp.int32)
                    v = plsc.load_gather(rb, [rows16, colv])
                    plsc.store_scatter(rb, [rows16, colv], v * sc)
                return 0
            lax.fori_loop(0, C // _LN, grp, 0)
            pltpu.sync_copy(rb, acc.at[dstv.at[j]], add=True)

        def pair(i, _):
            j0 = i * 2
            process(j0, rb0, gs0)

            @pl.when(j0 + 2 < NCH)
            def _():
                pltpu.async_copy(h_hbm.at[gidxv.at[j0 + 2]], rb0, gs0)

            process(j0 + 1, rb1, gs1)

            @pl.when(j0 + 3 < NCH)
            def _():
                pltpu.async_copy(h_hbm.at[gidxv.at[j0 + 3]], rb1, gs1)
            return 0
        lax.fori_loop(0, NCH // 2, pair, 0)
        if NCH % 2:
            process(NCH - 1, rb0, gs0)

        plsc.subcore_barrier()
        for k in range(NROW // ZR):
            rs = s * NROW + k * ZR
            pltpu.sync_copy(acc.at[pl.ds(rs, ZR)], out_hbm.at[c, pl.ds(rs, ZR)])

    return s2


def _t1(X, W):
    N, D = X.shape
    R = W.shape[0]
    HD = D // 2
    BN = 2000

    def body(x_ref, w_ref, o_ref):
        acc = jnp.dot(x_ref[...], w_ref[0], preferred_element_type=jnp.float32)
        o_ref[0] = acc[:, :HD]
        o_ref[1] = acc[:, HD:]

    return pl.pallas_call(
        body,
        grid=(N // BN, R),
        in_specs=[
            pl.BlockSpec((BN, D), lambda i, r: (i, 0)),
            pl.BlockSpec((1, D, D), lambda i, r: (r, 0, 0)),
        ],
        out_specs=pl.BlockSpec((2, BN, HD),
                               lambda i, r, _n=N // BN: (0, r * _n + i, 0)),
        out_shape=jax.ShapeDtypeStruct((2, R * N, HD), jnp.float32),
    )(X, W)


def _t3(X, eacc, Wself):
    N, D = X.shape
    HD = D // 2
    BN = 2000

    def body(x_ref, e_ref, ws_ref, o_ref):
        e = jnp.concatenate([e_ref[0], e_ref[1]], axis=1)
        o_ref[...] = jnp.maximum(
            jnp.dot(x_ref[...], ws_ref[...],
                    preferred_element_type=jnp.float32) + e, 0.0)

    return pl.pallas_call(
        body,
        grid=(N // BN,),
        in_specs=[
            pl.BlockSpec((BN, D), lambda i: (i, 0)),
            pl.BlockSpec((2, BN, HD), lambda i: (0, i, 0)),
            pl.BlockSpec((D, D), lambda i: (0, 0)),
        ],
        out_specs=pl.BlockSpec((BN, D), lambda i: (i, 0)),
        out_shape=jax.ShapeDtypeStruct((N, D), jnp.float32),
    )(X, eacc, Wself)


def kernel(X, edge_index, edge_type, W0, Wself0, W1, Wself1):
    N, D = X.shape
    R = W0.shape[0]
    E = edge_type.shape[0]
    RN = R * N
    HD = D // 2
    NP = ((N + 1023) // 1024) * 1024
    NH = NP // 2

    src = edge_index[0].astype(jnp.int32)
    dst = edge_index[1].astype(jnp.int32)
    et = edge_type.astype(jnp.int32)
    comb = et * N + dst
    gidx = et * N + src

    gidx2 = jnp.stack([gidx, gidx + RN]).reshape(2, _NS, E // (_NS * 80), 80)

    invtab = _p0a_build(E, RN)(comb)
    inve = _p0b_build(E, RN)(comb.reshape(_NW, E // (_NW * 40), 40), invtab)

    def half(p):
        inh = (dst >= p * NH) & (dst < (p + 1) * NH)
        dstp = jnp.clip(dst - p * NH, 0, NH - 1)
        invp = jnp.where(inh, inve, 0.0)
        return (dstp.reshape(_NS, E // (_NS * 80), 80),
                invp.reshape(_NS, E // (_NS * 80), 80))

    halves = [half(0), half(1)]
    s2 = _s2_build(NH, E, HD)

    def layer(h, W, Wself):
        hrel = _t1(h, W).reshape(2 * RN, HD)
        eh = [s2(hrel, gidx2, d3, i3) for d3, i3 in halves]
        eacc = jnp.concatenate([eh[0], eh[1]], axis=1)[:, :N]
        return _t3(h, eacc, Wself)

    h = layer(X, W0, Wself0)
    return layer(h, W1, Wself1)


# speedup vs baseline: 1.2472x; 1.2472x over previous
import functools

import jax
import jax.numpy as jnp
from jax import lax
from jax.experimental import pallas as pl
from jax.experimental.pallas import tpu as pltpu
from jax.experimental.pallas import tpu_sc as plsc

_NC = 2
_NS = 16
_LN = 16
_NW = _NC * _NS


def _mesh():
    return plsc.VectorSubcoreMesh(core_axis_name="c", subcore_axis_name="s")


def _p0a_build(E, RN):
    SEG = RN // _NW
    SEGP = ((SEG + 15) // 16) * 16
    CB = 2000

    @functools.partial(
        pl.kernel,
        out_type=jax.ShapeDtypeStruct((RN,), jnp.float32),
        mesh=_mesh(),
        compiler_params=pltpu.CompilerParams(needs_layout_passes=False),
        scratch_types=[
            pltpu.VMEM((SEGP,), jnp.float32),
            pltpu.VMEM((CB,), jnp.int32),
        ],
    )
    def p0a(comb_hbm, invtab_hbm, degv, cbuf):
        c = lax.axis_index("c")
        s = lax.axis_index("s")
        w = c * _NS + s
        lo = w * SEG
        zero = jnp.zeros((_LN,), jnp.float32)
        ones = jnp.ones((_LN,), jnp.float32)

        def zbody(i, _):
            degv[pl.ds(i * _LN, _LN)] = zero
            return 0
        lax.fori_loop(0, SEGP // _LN, zbody, 0)

        def chunk(j, _):
            pltpu.sync_copy(comb_hbm.at[pl.ds(j * CB, CB)], cbuf)

            def inner(k, _):
                cv = cbuf[pl.ds(k * _LN, _LN)]
                lv = cv - lo
                m = (lv >= 0) & (lv < SEG)
                lvc = jnp.clip(lv, 0, SEGP - 1)
                plsc.addupdate_scatter(degv, [lvc], ones, mask=m)
                return 0
            lax.fori_loop(0, CB // _LN, inner, 0)
            return 0
        lax.fori_loop(0, E // CB, chunk, 0)

        def ibody(i, _):
            dv = degv[pl.ds(i * _LN, _LN)]
            degv[pl.ds(i * _LN, _LN)] = 1.0 / jnp.maximum(dv, 1.0)
            return 0
        lax.fori_loop(0, SEGP // _LN, ibody, 0)

        pltpu.sync_copy(degv.at[pl.ds(0, SEG)], invtab_hbm.at[pl.ds(lo, SEG)])

    return p0a


def _p0b_build(E, RN):
    CW = 40
    NCH = E // (_NW * CW)

    @functools.partial(
        pl.kernel,
        out_type=jax.ShapeDtypeStruct((E,), jnp.float32),
        mesh=_mesh(),
        compiler_params=pltpu.CompilerParams(needs_layout_passes=False),
        scratch_types=[
            pltpu.VMEM((NCH, CW), jnp.int32),
            pltpu.VMEM((CW,), jnp.float32),
            pltpu.VMEM((CW,), jnp.float32),
            pltpu.SemaphoreType.DMA,
            pltpu.SemaphoreType.DMA,
        ],
    )
    def p0b(comb3_hbm, invtab_hbm, inve_hbm, cidx, ib0, ib1, s0, s1):
        c = lax.axis_index("c")
        s = lax.axis_index("s")
        w = c * _NS + s
        base = w * NCH
        pltpu.sync_copy(comb3_hbm.at[w], cidx)

        pltpu.async_copy(invtab_hbm.at[cidx.at[0]], ib0, s0)
        pltpu.async_copy(invtab_hbm.at[cidx.at[1]], ib1, s1)

        def step(j, ib, sem):
            pltpu.make_async_copy(invtab_hbm.at[cidx.at[j]], ib, sem).wait()
            pltpu.sync_copy(ib, inve_hbm.at[pl.ds((base + j) * CW, CW)])

        def pair(i, _):
            j0 = i * 2
            step(j0, ib0, s0)

            @pl.when(j0 + 2 < NCH)
            def _():
                pltpu.async_copy(invtab_hbm.at[cidx.at[j0 + 2]], ib0, s0)

            step(j0 + 1, ib1, s1)

            @pl.when(j0 + 3 < NCH)
            def _():
                pltpu.async_copy(invtab_hbm.at[cidx.at[j0 + 3]], ib1, s1)
            return 0
        lax.fori_loop(0, NCH // 2, pair, 0)
        if NCH % 2:
            step(NCH - 1, ib0, s0)

    return p0b


def _s2_build(NP, E, RN, HD):
    C = 80
    NCHT = E // C
    NROW = NP // _NS

    @functools.partial(
        pl.kernel,
        out_type=jax.ShapeDtypeStruct((_NC, NP, HD), jnp.float32),
        mesh=_mesh(),
        compiler_params=pltpu.CompilerParams(needs_layout_passes=False),
        scratch_types=[
            pltpu.VMEM((C,), jnp.int32),
            pltpu.VMEM((C,), jnp.int32),
            pltpu.VMEM((C,), jnp.int32),
            pltpu.VMEM((C,), jnp.int32),
            pltpu.VMEM((C,), jnp.float32),
            pltpu.VMEM((C,), jnp.float32),
            pltpu.VMEM((C, HD), jnp.float32),
            pltpu.VMEM((C, HD), jnp.float32),
            pltpu.VMEM((NROW, HD), jnp.float32),
            pltpu.VMEM((32,), jnp.int32),
            pltpu.SemaphoreType.DMA,
            pltpu.SemaphoreType.DMA,
        ],
    )
    def s2(h_hbm, gidx_hbm, dst_hbm, inv_hbm, bnd_hbm, out_hbm,
           cb0, cb1, db0, db1, ib0, ib1, rb0, rb1, acc, bv, gs0, gs1):
        c = lax.axis_index("c")
        s = lax.axis_index("s")
        lo = s * NROW
        coff = c * RN
        zero = jnp.zeros((_LN,), jnp.float32)
        iota = lax.iota(jnp.int32, _LN)

        pltpu.sync_copy(bnd_hbm, bv)
        v0 = bv[pl.ds(0, _LN)]
        v1 = bv[pl.ds(_LN, _LN)]
        zi = jnp.zeros((_LN,), jnp.int32)
        b0 = jnp.sum(jnp.where(iota == s, v0, zi))
        b1 = (jnp.sum(jnp.where(iota == s + 1, v0, zi))
              + jnp.sum(jnp.where(iota + _LN == s + 1, v1, zi)))
        c0 = b0 // C
        c1 = (b1 + C - 1) // C
        n = c1 - c0

        def zrow(i, _):
            def zcol(g, _):
                acc[i, pl.ds(g * _LN, _LN)] = zero
                return 0
            lax.fori_loop(0, HD // _LN, zcol, 0)
            return 0
        lax.fori_loop(0, NROW, zrow, 0)

        def fetch(k, cb, db, ib, rb, gs):
            o = pl.multiple_of(k * C, 8)
            pltpu.sync_copy(gidx_hbm.at[pl.ds(o, C)], cb)

            def addoff(g, _):
                cb[pl.ds(g * _LN, _LN)] = cb[pl.ds(g * _LN, _LN)] + coff
                return 0
            lax.fori_loop(0, C // _LN, addoff, 0)
            pltpu.sync_copy(dst_hbm.at[pl.ds(o, C)], db)
            pltpu.sync_copy(inv_hbm.at[pl.ds(o, C)], ib)
            pltpu.async_copy(h_hbm.at[cb], rb, gs)

        @pl.when(n > 0)
        def _():
            fetch(c0, cb0, db0, ib0, rb0, gs0)

        @pl.when(n > 1)
        def _():
            fetch(c0 + 1, cb1, db1, ib1, rb1, gs1)

        def process(cb, db, ib, rb, gs):
            pltpu.make_async_copy(h_hbm.at[cb], rb, gs).wait()

            def grp(g, _):
                dv = db[pl.ds(g * _LN, _LN)] - lo
                sc = ib[pl.ds(g * _LN, _LN)]
                m = (dv >= 0) & (dv < NROW)
                dvc = jnp.clip(dv, 0, NROW - 1)
                rows16 = iota + g * _LN
                for col in range(HD):
                    colv = jnp.full((_LN,), col, jnp.int32)
                    v = plsc.load_gather(rb, [rows16, colv])
                    plsc.addupdate_scatter(acc, [dvc, colv], v * sc, mask=m)
                return 0
            lax.fori_loop(0, C // _LN, grp, 0)

        def pair(i, _):
            j0 = c0 + i * 2
            process(cb0, db0, ib0, rb0, gs0)

            @pl.when(j0 + 2 < c1)
            def _():
                fetch(j0 + 2, cb0, db0, ib0, rb0, gs0)

            process(cb1, db1, ib1, rb1, gs1)

            @pl.when(j0 + 3 < c1)
            def _():
                fetch(j0 + 3, cb1, db1, ib1, rb1, gs1)
            return 0
        lax.fori_loop(0, n // 2, pair, 0)

        @pl.when(n % 2 == 1)
        def _():
            process(cb0, db0, ib0, rb0, gs0)

        pltpu.sync_copy(acc, out_hbm.at[c, pl.ds(lo, NROW)])

    return s2


def _t1(X, W):
    N, D = X.shape
    R = W.shape[0]
    HD = D // 2
    BN = 2000

    def body(x_ref, w_ref, o_ref):
        acc = jnp.dot(x_ref[...], w_ref[0], preferred_element_type=jnp.float32)
        o_ref[0] = acc[:, :HD]
        o_ref[1] = acc[:, HD:]

    return pl.pallas_call(
        body,
        grid=(N // BN, R),
        in_specs=[
            pl.BlockSpec((BN, D), lambda i, r: (i, 0)),
            pl.BlockSpec((1, D, D), lambda i, r: (r, 0, 0)),
        ],
        out_specs=pl.BlockSpec((2, BN, HD),
                               lambda i, r, _n=N // BN: (0, r * _n + i, 0)),
        out_shape=jax.ShapeDtypeStruct((2, R * N, HD), jnp.float32),
    )(X, W)


def _t3(X, eacc, Wself):
    N, D = X.shape
    HD = D // 2
    BN = 2000

    def body(x_ref, e_ref, ws_ref, o_ref):
        e = jnp.concatenate([e_ref[0], e_ref[1]], axis=1)
        o_ref[...] = jnp.maximum(
            jnp.dot(x_ref[...], ws_ref[...],
                    preferred_element_type=jnp.float32) + e, 0.0)

    return pl.pallas_call(
        body,
        grid=(N // BN,),
        in_specs=[
            pl.BlockSpec((BN, D), lambda i: (i, 0)),
            pl.BlockSpec((2, BN, HD), lambda i: (0, i, 0)),
            pl.BlockSpec((D, D), lambda i: (0, 0)),
        ],
        out_specs=pl.BlockSpec((BN, D), lambda i: (i, 0)),
        out_shape=jax.ShapeDtypeStruct((N, D), jnp.float32),
    )(X, eacc, Wself)


def kernel(X, edge_index, edge_type, W0, Wself0, W1, Wself1):
    N, D = X.shape
    R = W0.shape[0]
    E = edge_type.shape[0]
    RN = R * N
    HD = D // 2
    NP = ((N + 1023) // 1024) * 1024
    NH = NP // 2

    src = edge_index[0].astype(jnp.int32)
    dst = edge_index[1].astype(jnp.int32)
    et = edge_type.astype(jnp.int32)

    order = jnp.argsort(dst)
    dsts = dst[order]
    combs = et[order] * N + dsts
    gidxs = (et * N + src)[order]

    NROW = NP // _NS
    bnd = jnp.searchsorted(
        dsts, jnp.arange(_NS + 1) * NROW).astype(jnp.int32)
    bnd32 = jnp.concatenate([bnd, jnp.full((32 - _NS - 1,), E, jnp.int32)])

    invtab = _p0a_build(E, RN)(combs)
    invs = _p0b_build(E, RN)(combs.reshape(_NW, E // (_NW * 40), 40), invtab)

    s2 = _s2_build(NP, E, RN, HD)

    def layer(h, W, Wself):
        hrel = _t1(h, W).reshape(2 * RN, HD)
        eacc = s2(hrel, gidxs, dsts, invs, bnd32)[:, :N]
        return _t3(h, eacc, Wself)

    h = layer(X, W0, Wself0)
    return layer(h, W1, Wself1)


# speedup vs baseline: 3.3945x; 2.7218x over previous
import functools

import jax
import jax.numpy as jnp
from jax import lax
from jax.experimental import pallas as pl
from jax.experimental.pallas import tpu as pltpu
from jax.experimental.pallas import tpu_sc as plsc

_NC = 2
_NS = 16
_LN = 16
_NW = _NC * _NS


def _mesh():
    return plsc.VectorSubcoreMesh(core_axis_name="c", subcore_axis_name="s")


def _p0a_build(E, RN):
    SEG = RN // _NW
    SEGP = ((SEG + 15) // 16) * 16
    CB = 2000

    @functools.partial(
        pl.kernel,
        out_type=jax.ShapeDtypeStruct((RN,), jnp.float32),
        mesh=_mesh(),
        compiler_params=pltpu.CompilerParams(needs_layout_passes=False),
        scratch_types=[
            pltpu.VMEM((SEGP,), jnp.float32),
            pltpu.VMEM((CB,), jnp.int32),
        ],
    )
    def p0a(comb_hbm, invtab_hbm, degv, cbuf):
        c = lax.axis_index("c")
        s = lax.axis_index("s")
        w = c * _NS + s
        lo = w * SEG
        zero = jnp.zeros((_LN,), jnp.float32)
        ones = jnp.ones((_LN,), jnp.float32)

        def zbody(i, _):
            degv[pl.ds(i * _LN, _LN)] = zero
            return 0
        lax.fori_loop(0, SEGP // _LN, zbody, 0)

        def chunk(j, _):
            pltpu.sync_copy(comb_hbm.at[pl.ds(j * CB, CB)], cbuf)

            def inner(k, _):
                cv = cbuf[pl.ds(k * _LN, _LN)]
                lv = cv - lo
                m = (lv >= 0) & (lv < SEG)
                lvc = jnp.clip(lv, 0, SEGP - 1)
                plsc.addupdate_scatter(degv, [lvc], ones, mask=m)
                return 0
            lax.fori_loop(0, CB // _LN, inner, 0)
            return 0
        lax.fori_loop(0, E // CB, chunk, 0)

        def ibody(i, _):
            dv = degv[pl.ds(i * _LN, _LN)]
            degv[pl.ds(i * _LN, _LN)] = 1.0 / jnp.maximum(dv, 1.0)
            return 0
        lax.fori_loop(0, SEGP // _LN, ibody, 0)

        pltpu.sync_copy(degv.at[pl.ds(0, SEG)], invtab_hbm.at[pl.ds(lo, SEG)])

    return p0a


def _p0b_build(E, RN):
    CW = 40
    NCH = E // (_NW * CW)

    @functools.partial(
        pl.kernel,
        out_type=jax.ShapeDtypeStruct((E,), jnp.float32),
        mesh=_mesh(),
        compiler_params=pltpu.CompilerParams(needs_layout_passes=False),
        scratch_types=[
            pltpu.VMEM((NCH, CW), jnp.int32),
            pltpu.VMEM((CW,), jnp.float32),
            pltpu.VMEM((CW,), jnp.float32),
            pltpu.SemaphoreType.DMA,
            pltpu.SemaphoreType.DMA,
        ],
    )
    def p0b(comb3_hbm, invtab_hbm, inve_hbm, cidx, ib0, ib1, s0, s1):
        c = lax.axis_index("c")
        s = lax.axis_index("s")
        w = c * _NS + s
        base = w * NCH
        pltpu.sync_copy(comb3_hbm.at[w], cidx)

        pltpu.async_copy(invtab_hbm.at[cidx.at[0]], ib0, s0)
        pltpu.async_copy(invtab_hbm.at[cidx.at[1]], ib1, s1)

        def step(j, ib, sem):
            pltpu.make_async_copy(invtab_hbm.at[cidx.at[j]], ib, sem).wait()
            pltpu.sync_copy(ib, inve_hbm.at[pl.ds((base + j) * CW, CW)])

        def pair(i, _):
            j0 = i * 2
            step(j0, ib0, s0)

            @pl.when(j0 + 2 < NCH)
            def _():
                pltpu.async_copy(invtab_hbm.at[cidx.at[j0 + 2]], ib0, s0)

            step(j0 + 1, ib1, s1)

            @pl.when(j0 + 3 < NCH)
            def _():
                pltpu.async_copy(invtab_hbm.at[cidx.at[j0 + 3]], ib1, s1)
            return 0
        lax.fori_loop(0, NCH // 2, pair, 0)
        if NCH % 2:
            step(NCH - 1, ib0, s0)

    return p0b


def _s2_build(NP, E, RN, HD):
    C = 80
    NCHT = E // C
    NROW = NP // _NS

    @functools.partial(
        pl.kernel,
        out_type=jax.ShapeDtypeStruct((_NC, NP, HD), jnp.float32),
        mesh=_mesh(),
        compiler_params=pltpu.CompilerParams(needs_layout_passes=False),
        scratch_types=[
            pltpu.VMEM((C,), jnp.int32),
            pltpu.VMEM((C,), jnp.int32),
            pltpu.VMEM((C,), jnp.int32),
            pltpu.VMEM((C,), jnp.int32),
            pltpu.VMEM((C,), jnp.float32),
            pltpu.VMEM((C,), jnp.float32),
            pltpu.VMEM((C, HD), jnp.float32),
            pltpu.VMEM((C, HD), jnp.float32),
            pltpu.VMEM((NROW, HD), jnp.float32),
            pltpu.VMEM((32,), jnp.int32),
            pltpu.SemaphoreType.DMA,
            pltpu.SemaphoreType.DMA,
        ],
    )
    def s2(h_hbm, gidx_hbm, dst_hbm, inv_hbm, bnd_hbm, out_hbm,
           cb0, cb1, db0, db1, ib0, ib1, rb0, rb1, acc, bv, gs0, gs1):
        c = lax.axis_index("c")
        s = lax.axis_index("s")
        lo = s * NROW
        coff = c * RN
        zero = jnp.zeros((_LN,), jnp.float32)
        iota = lax.iota(jnp.int32, _LN)

        pltpu.sync_copy(bnd_hbm, bv)
        v0 = bv[pl.ds(0, _LN)]
        v1 = bv[pl.ds(_LN, _LN)]
        zi = jnp.zeros((_LN,), jnp.int32)
        b0 = jnp.sum(jnp.where(iota == s, v0, zi))
        b1 = (jnp.sum(jnp.where(iota == s + 1, v0, zi))
              + jnp.sum(jnp.where(iota + _LN == s + 1, v1, zi)))
        c0 = b0 // C
        c1 = (b1 + C - 1) // C
        n = c1 - c0

        def zrow(i, _):
            def zcol(g, _):
                acc[i, pl.ds(g * _LN, _LN)] = zero
                return 0
            lax.fori_loop(0, HD // _LN, zcol, 0)
            return 0
        lax.fori_loop(0, NROW, zrow, 0)

        def fetch(k, cb, db, ib, rb, gs):
            o = pl.multiple_of(k * C, 8)
            pltpu.sync_copy(gidx_hbm.at[pl.ds(o, C)], cb)

            def addoff(g, _):
                cb[pl.ds(g * _LN, _LN)] = cb[pl.ds(g * _LN, _LN)] + coff
                return 0
            lax.fori_loop(0, C // _LN, addoff, 0)
            pltpu.sync_copy(dst_hbm.at[pl.ds(o, C)], db)
            pltpu.sync_copy(inv_hbm.at[pl.ds(o, C)], ib)
            pltpu.async_copy(h_hbm.at[cb], rb, gs)

        @pl.when(n > 0)
        def _():
            fetch(c0, cb0, db0, ib0, rb0, gs0)

        @pl.when(n > 1)
        def _():
            fetch(c0 + 1, cb1, db1, ib1, rb1, gs1)

        def process(cb, db, ib, rb, gs):
            pltpu.make_async_copy(h_hbm.at[cb], rb, gs).wait()
            zf = jnp.zeros((_LN,), jnp.float32)
            zi2 = jnp.zeros((_LN,), jnp.int32)

            def grp(g, _):
                dv = db[pl.ds(g * _LN, _LN)] - lo
                m = (dv >= 0) & (dv < NROW)
                dvc = jnp.clip(dv, 0, NROW - 1)
                scm = jnp.where(m, ib[pl.ds(g * _LN, _LN)], zf)
                for e in range(_LN):
                    lane = iota == e
                    dve = jnp.sum(jnp.where(lane, dvc, zi2))
                    se = jnp.sum(jnp.where(lane, scm, zf))
                    r = g * _LN + e
                    for g2 in range(HD // _LN):
                        v = rb[r, pl.ds(g2 * _LN, _LN)]
                        acc[dve, pl.ds(g2 * _LN, _LN)] = (
                            acc[dve, pl.ds(g2 * _LN, _LN)] + v * se)
                return 0
            lax.fori_loop(0, C // _LN, grp, 0)

        def pair(i, _):
            j0 = c0 + i * 2
            process(cb0, db0, ib0, rb0, gs0)

            @pl.when(j0 + 2 < c1)
            def _():
                fetch(j0 + 2, cb0, db0, ib0, rb0, gs0)

            process(cb1, db1, ib1, rb1, gs1)

            @pl.when(j0 + 3 < c1)
            def _():
                fetch(j0 + 3, cb1, db1, ib1, rb1, gs1)
            return 0
        lax.fori_loop(0, n // 2, pair, 0)

        @pl.when(n % 2 == 1)
        def _():
            process(cb0, db0, ib0, rb0, gs0)

        pltpu.sync_copy(acc, out_hbm.at[c, pl.ds(lo, NROW)])

    return s2


def _t1(X, W):
    N, D = X.shape
    R = W.shape[0]
    HD = D // 2
    BN = 2000

    def body(x_ref, w_ref, o_ref):
        acc = jnp.dot(x_ref[...], w_ref[0], preferred_element_type=jnp.float32)
        o_ref[0] = acc[:, :HD]
        o_ref[1] = acc[:, HD:]

    return pl.pallas_call(
        body,
        grid=(N // BN, R),
        in_specs=[
            pl.BlockSpec((BN, D), lambda i, r: (i, 0)),
            pl.BlockSpec((1, D, D), lambda i, r: (r, 0, 0)),
        ],
        out_specs=pl.BlockSpec((2, BN, HD),
                               lambda i, r, _n=N // BN: (0, r * _n + i, 0)),
        out_shape=jax.ShapeDtypeStruct((2, R * N, HD), jnp.float32),
    )(X, W)


def _t3(X, eacc, Wself):
    N, D = X.shape
    HD = D // 2
    BN = 2000

    def body(x_ref, e_ref, ws_ref, o_ref):
        e = jnp.concatenate([e_ref[0], e_ref[1]], axis=1)
        o_ref[...] = jnp.maximum(
            jnp.dot(x_ref[...], ws_ref[...],
                    preferred_element_type=jnp.float32) + e, 0.0)

    return pl.pallas_call(
        body,
        grid=(N // BN,),
        in_specs=[
            pl.BlockSpec((BN, D), lambda i: (i, 0)),
            pl.BlockSpec((2, BN, HD), lambda i: (0, i, 0)),
            pl.BlockSpec((D, D), lambda i: (0, 0)),
        ],
        out_specs=pl.BlockSpec((BN, D), lambda i: (i, 0)),
        out_shape=jax.ShapeDtypeStruct((N, D), jnp.float32),
    )(X, eacc, Wself)


def kernel(X, edge_index, edge_type, W0, Wself0, W1, Wself1):
    N, D = X.shape
    R = W0.shape[0]
    E = edge_type.shape[0]
    RN = R * N
    HD = D // 2
    NP = ((N + 1023) // 1024) * 1024
    NH = NP // 2

    src = edge_index[0].astype(jnp.int32)
    dst = edge_index[1].astype(jnp.int32)
    et = edge_type.astype(jnp.int32)

    order = jnp.argsort(dst)
    dsts = dst[order]
    combs = et[order] * N + dsts
    gidxs = (et * N + src)[order]

    NROW = NP // _NS
    bnd = jnp.searchsorted(
        dsts, jnp.arange(_NS + 1) * NROW).astype(jnp.int32)
    bnd32 = jnp.concatenate([bnd, jnp.full((32 - _NS - 1,), E, jnp.int32)])

    invtab = _p0a_build(E, RN)(combs)
    invs = _p0b_build(E, RN)(combs.reshape(_NW, E // (_NW * 40), 40), invtab)

    s2 = _s2_build(NP, E, RN, HD)

    def layer(h, W, Wself):
        hrel = _t1(h, W).reshape(2 * RN, HD)
        eacc = s2(hrel, gidxs, dsts, invs, bnd32)[:, :N]
        return _t3(h, eacc, Wself)

    h = layer(X, W0, Wself0)
    return layer(h, W1, Wself1)


# speedup vs baseline: 3.8138x; 1.1235x over previous
import functools

import jax
import jax.numpy as jnp
from jax import lax
from jax.experimental import pallas as pl
from jax.experimental.pallas import tpu as pltpu
from jax.experimental.pallas import tpu_sc as plsc

_NC = 2
_NS = 16
_LN = 16
_NW = _NC * _NS


def _mesh():
    return plsc.VectorSubcoreMesh(core_axis_name="c", subcore_axis_name="s")


def _p0a_build(E, RN):
    SEG = RN // _NW
    SEGP = ((SEG + 15) // 16) * 16
    CB = 8000

    @functools.partial(
        pl.kernel,
        out_type=jax.ShapeDtypeStruct((RN,), jnp.float32),
        mesh=_mesh(),
        compiler_params=pltpu.CompilerParams(needs_layout_passes=False),
        scratch_types=[
            pltpu.VMEM((SEGP,), jnp.float32),
            pltpu.VMEM((CB,), jnp.int32),
        ],
    )
    def p0a(comb_hbm, invtab_hbm, degv, cbuf):
        c = lax.axis_index("c")
        s = lax.axis_index("s")
        w = c * _NS + s
        lo = w * SEG
        zero = jnp.zeros((_LN,), jnp.float32)
        ones = jnp.ones((_LN,), jnp.float32)

        def zbody(i, _):
            degv[pl.ds(i * _LN, _LN)] = zero
            return 0
        lax.fori_loop(0, SEGP // _LN, zbody, 0)

        def chunk(j, _):
            pltpu.sync_copy(comb_hbm.at[pl.ds(j * CB, CB)], cbuf)

            def inner(k, _):
                cv = cbuf[pl.ds(k * _LN, _LN)]
                lv = cv - lo
                m = (lv >= 0) & (lv < SEG)
                lvc = jnp.clip(lv, 0, SEGP - 1)
                plsc.addupdate_scatter(degv, [lvc], ones, mask=m)
                return 0
            lax.fori_loop(0, CB // _LN, inner, 0)
            return 0
        lax.fori_loop(0, E // CB, chunk, 0)

        def ibody(i, _):
            dv = degv[pl.ds(i * _LN, _LN)]
            degv[pl.ds(i * _LN, _LN)] = 1.0 / jnp.maximum(dv, 1.0)
            return 0
        lax.fori_loop(0, SEGP // _LN, ibody, 0)

        pltpu.sync_copy(degv.at[pl.ds(0, SEG)], invtab_hbm.at[pl.ds(lo, SEG)])

    return p0a


def _p0b_build(E, RN):
    CW = 40
    NCH = E // (_NW * CW)

    @functools.partial(
        pl.kernel,
        out_type=jax.ShapeDtypeStruct((E,), jnp.float32),
        mesh=_mesh(),
        compiler_params=pltpu.CompilerParams(needs_layout_passes=False),
        scratch_types=[
            pltpu.VMEM((NCH, CW), jnp.int32),
            pltpu.VMEM((CW,), jnp.float32),
            pltpu.VMEM((CW,), jnp.float32),
            pltpu.SemaphoreType.DMA,
            pltpu.SemaphoreType.DMA,
        ],
    )
    def p0b(comb3_hbm, invtab_hbm, inve_hbm, cidx, ib0, ib1, s0, s1):
        c = lax.axis_index("c")
        s = lax.axis_index("s")
        w = c * _NS + s
        base = w * NCH
        pltpu.sync_copy(comb3_hbm.at[w], cidx)

        pltpu.async_copy(invtab_hbm.at[cidx.at[0]], ib0, s0)
        pltpu.async_copy(invtab_hbm.at[cidx.at[1]], ib1, s1)

        def step(j, ib, sem):
            pltpu.make_async_copy(invtab_hbm.at[cidx.at[j]], ib, sem).wait()
            pltpu.sync_copy(ib, inve_hbm.at[pl.ds((base + j) * CW, CW)])

        def pair(i, _):
            j0 = i * 2
            step(j0, ib0, s0)

            @pl.when(j0 + 2 < NCH)
            def _():
                pltpu.async_copy(invtab_hbm.at[cidx.at[j0 + 2]], ib0, s0)

            step(j0 + 1, ib1, s1)

            @pl.when(j0 + 3 < NCH)
            def _():
                pltpu.async_copy(invtab_hbm.at[cidx.at[j0 + 3]], ib1, s1)
            return 0
        lax.fori_loop(0, NCH // 2, pair, 0)
        if NCH % 2:
            step(NCH - 1, ib0, s0)

    return p0b


def _s2_build(NP, E, RN, HD):
    C = 80
    NCHT = E // C
    NROW = NP // _NS

    @functools.partial(
        pl.kernel,
        out_type=jax.ShapeDtypeStruct((_NC, NP, HD), jnp.float32),
        mesh=_mesh(),
        compiler_params=pltpu.CompilerParams(needs_layout_passes=False),
        scratch_types=[
            pltpu.VMEM((3, C), jnp.int32),
            pltpu.VMEM((3, C), jnp.int32),
            pltpu.VMEM((C, HD), jnp.float32),
            pltpu.VMEM((C, HD), jnp.float32),
            pltpu.VMEM((NROW, HD), jnp.float32),
            pltpu.VMEM((32,), jnp.int32),
            pltpu.SemaphoreType.DMA,
            pltpu.SemaphoreType.DMA,
        ],
    )
    def s2(h_hbm, meta_hbm, bnd_hbm, out_hbm,
           mb0, mb1, rb0, rb1, acc, bv, gs0, gs1):
        c = lax.axis_index("c")
        s = lax.axis_index("s")
        lo = s * NROW
        coff = c * RN
        zero = jnp.zeros((_LN,), jnp.float32)
        iota = lax.iota(jnp.int32, _LN)

        pltpu.sync_copy(bnd_hbm, bv)
        v0 = bv[pl.ds(0, _LN)]
        v1 = bv[pl.ds(_LN, _LN)]
        zi = jnp.zeros((_LN,), jnp.int32)
        b0 = jnp.sum(jnp.where(iota == s, v0, zi))
        b1 = (jnp.sum(jnp.where(iota == s + 1, v0, zi))
              + jnp.sum(jnp.where(iota + _LN == s + 1, v1, zi)))
        c0 = b0 // C
        c1 = (b1 + C - 1) // C
        n = c1 - c0

        def zrow(i, _):
            def zcol(g, _):
                acc[i, pl.ds(g * _LN, _LN)] = zero
                return 0
            lax.fori_loop(0, HD // _LN, zcol, 0)
            return 0
        lax.fori_loop(0, NROW, zrow, 0)

        def fetch(k, mb, rb, gs):
            pltpu.sync_copy(meta_hbm.at[k], mb)

            def addoff(g, _):
                mb[0, pl.ds(g * _LN, _LN)] = mb[0, pl.ds(g * _LN, _LN)] + coff
                return 0
            lax.fori_loop(0, C // _LN, addoff, 0)
            pltpu.async_copy(h_hbm.at[mb.at[0]], rb, gs)

        @pl.when(n > 0)
        def _():
            fetch(c0, mb0, rb0, gs0)

        @pl.when(n > 1)
        def _():
            fetch(c0 + 1, mb1, rb1, gs1)

        def process(mb, rb, gs):
            pltpu.make_async_copy(h_hbm.at[mb.at[0]], rb, gs).wait()
            zf = jnp.zeros((_LN,), jnp.float32)
            zi2 = jnp.zeros((_LN,), jnp.int32)

            def grp(g, _):
                dv = mb[1, pl.ds(g * _LN, _LN)] - lo
                m = (dv >= 0) & (dv < NROW)
                dvc = jnp.clip(dv, 0, NROW - 1)
                iv = plsc.bitcast(mb[2, pl.ds(g * _LN, _LN)], jnp.float32)
                scm = jnp.where(m, iv, zf)
                for e in range(_LN):
                    lane = iota == e
                    dve = jnp.sum(jnp.where(lane, dvc, zi2))
                    se = jnp.sum(jnp.where(lane, scm, zf))
                    r = g * _LN + e
                    for g2 in range(HD // _LN):
                        v = rb[r, pl.ds(g2 * _LN, _LN)]
                        acc[dve, pl.ds(g2 * _LN, _LN)] = (
                            acc[dve, pl.ds(g2 * _LN, _LN)] + v * se)
                return 0
            lax.fori_loop(0, C // _LN, grp, 0)

        def pair(i, _):
            j0 = c0 + i * 2
            process(mb0, rb0, gs0)

            @pl.when(j0 + 2 < c1)
            def _():
                fetch(j0 + 2, mb0, rb0, gs0)

            process(mb1, rb1, gs1)

            @pl.when(j0 + 3 < c1)
            def _():
                fetch(j0 + 3, mb1, rb1, gs1)
            return 0
        lax.fori_loop(0, n // 2, pair, 0)

        @pl.when(n % 2 == 1)
        def _():
            process(mb0, rb0, gs0)

        pltpu.sync_copy(acc, out_hbm.at[c, pl.ds(lo, NROW)])

    return s2


def _t1(X, W):
    N, D = X.shape
    R = W.shape[0]
    HD = D // 2
    BN = 2000

    def body(x_ref, w_ref, o_ref):
        acc = jnp.dot(x_ref[...], w_ref[0], preferred_element_type=jnp.float32)
        o_ref[0] = acc[:, :HD]
        o_ref[1] = acc[:, HD:]

    return pl.pallas_call(
        body,
        grid=(N // BN, R),
        in_specs=[
            pl.BlockSpec((BN, D), lambda i, r: (i, 0)),
            pl.BlockSpec((1, D, D), lambda i, r: (r, 0, 0)),
        ],
        out_specs=pl.BlockSpec((2, BN, HD),
                               lambda i, r, _n=N // BN: (0, r * _n + i, 0)),
        out_shape=jax.ShapeDtypeStruct((2, R * N, HD), jnp.float32),
    )(X, W)


def _t3(X, eacc, Wself):
    N, D = X.shape
    HD = D // 2
    BN = 2000

    def body(x_ref, e_ref, ws_ref, o_ref):
        e = jnp.concatenate([e_ref[0], e_ref[1]], axis=1)
        o_ref[...] = jnp.maximum(
            jnp.dot(x_ref[...], ws_ref[...],
                    preferred_element_type=jnp.float32) + e, 0.0)

    return pl.pallas_call(
        body,
        grid=(N // BN,),
        in_specs=[
            pl.BlockSpec((BN, D), lambda i: (i, 0)),
            pl.BlockSpec((2, BN, HD), lambda i: (0, i, 0)),
            pl.BlockSpec((D, D), lambda i: (0, 0)),
        ],
        out_specs=pl.BlockSpec((BN, D), lambda i: (i, 0)),
        out_shape=jax.ShapeDtypeStruct((N, D), jnp.float32),
    )(X, eacc, Wself)


def kernel(X, edge_index, edge_type, W0, Wself0, W1, Wself1):
    N, D = X.shape
    R = W0.shape[0]
    E = edge_type.shape[0]
    RN = R * N
    HD = D // 2
    NP = ((N + 1023) // 1024) * 1024
    NH = NP // 2

    src = edge_index[0].astype(jnp.int32)
    dst = edge_index[1].astype(jnp.int32)
    et = edge_type.astype(jnp.int32)

    order = jnp.argsort(dst)
    dsts = dst[order]
    combs = et[order] * N + dsts
    gidxs = (et * N + src)[order]

    NROW = NP // _NS
    bnd = jnp.searchsorted(
        dsts, jnp.arange(_NS + 1) * NROW).astype(jnp.int32)
    bnd32 = jnp.concatenate([bnd, jnp.full((32 - _NS - 1,), E, jnp.int32)])

    invtab = _p0a_build(E, RN)(combs)
    invs = _p0b_build(E, RN)(combs.reshape(_NW, E // (_NW * 40), 40), invtab)

    meta = jnp.stack(
        [gidxs, dsts, jax.lax.bitcast_convert_type(invs, jnp.int32)]
    ).reshape(3, E // 80, 80).transpose(1, 0, 2)

    s2 = _s2_build(NP, E, RN, HD)

    def layer(h, W, Wself):
        hrel = _t1(h, W).reshape(2 * RN, HD)
        eacc = s2(hrel, meta, bnd32)[:, :N]
        return _t3(h, eacc, Wself)

    h = layer(X, W0, Wself0)
    return layer(h, W1, Wself1)


# speedup vs baseline: 3.8147x; 1.0003x over previous
import functools

import jax
import jax.numpy as jnp
from jax import lax
from jax.experimental import pallas as pl
from jax.experimental.pallas import tpu as pltpu
from jax.experimental.pallas import tpu_sc as plsc

_NC = 2
_NS = 16
_LN = 16
_NW = _NC * _NS


def _mesh():
    return plsc.VectorSubcoreMesh(core_axis_name="c", subcore_axis_name="s")


def _p0a_build(E, RN):
    SEG = RN // _NW
    SEGP = ((SEG + 15) // 16) * 16
    CB = 8000

    @functools.partial(
        pl.kernel,
        out_type=jax.ShapeDtypeStruct((RN,), jnp.float32),
        mesh=_mesh(),
        compiler_params=pltpu.CompilerParams(needs_layout_passes=False),
        scratch_types=[
            pltpu.VMEM((SEGP,), jnp.float32),
            pltpu.VMEM((CB,), jnp.int32),
        ],
    )
    def p0a(comb_hbm, invtab_hbm, degv, cbuf):
        c = lax.axis_index("c")
        s = lax.axis_index("s")
        w = c * _NS + s
        lo = w * SEG
        zero = jnp.zeros((_LN,), jnp.float32)
        ones = jnp.ones((_LN,), jnp.float32)

        def zbody(i, _):
            degv[pl.ds(i * _LN, _LN)] = zero
            return 0
        lax.fori_loop(0, SEGP // _LN, zbody, 0)

        def chunk(j, _):
            pltpu.sync_copy(comb_hbm.at[pl.ds(j * CB, CB)], cbuf)

            def inner(k, _):
                cv = cbuf[pl.ds(k * _LN, _LN)]
                lv = cv - lo
                m = (lv >= 0) & (lv < SEG)
                lvc = jnp.clip(lv, 0, SEGP - 1)
                plsc.addupdate_scatter(degv, [lvc], ones, mask=m)
                return 0
            lax.fori_loop(0, CB // _LN, inner, 0)
            return 0
        lax.fori_loop(0, E // CB, chunk, 0)

        def ibody(i, _):
            dv = degv[pl.ds(i * _LN, _LN)]
            degv[pl.ds(i * _LN, _LN)] = 1.0 / jnp.maximum(dv, 1.0)
            return 0
        lax.fori_loop(0, SEGP // _LN, ibody, 0)

        pltpu.sync_copy(degv.at[pl.ds(0, SEG)], invtab_hbm.at[pl.ds(lo, SEG)])

    return p0a


def _p0b_build(E, RN):
    CW = 40
    NCH = E // (_NW * CW)

    @functools.partial(
        pl.kernel,
        out_type=jax.ShapeDtypeStruct((E,), jnp.float32),
        mesh=_mesh(),
        compiler_params=pltpu.CompilerParams(needs_layout_passes=False),
        scratch_types=[
            pltpu.VMEM((NCH, CW), jnp.int32),
            pltpu.VMEM((CW,), jnp.float32),
            pltpu.VMEM((CW,), jnp.float32),
            pltpu.SemaphoreType.DMA,
            pltpu.SemaphoreType.DMA,
        ],
    )
    def p0b(comb3_hbm, invtab_hbm, inve_hbm, cidx, ib0, ib1, s0, s1):
        c = lax.axis_index("c")
        s = lax.axis_index("s")
        w = c * _NS + s
        base = w * NCH
        pltpu.sync_copy(comb3_hbm.at[w], cidx)

        pltpu.async_copy(invtab_hbm.at[cidx.at[0]], ib0, s0)
        pltpu.async_copy(invtab_hbm.at[cidx.at[1]], ib1, s1)

        def step(j, ib, sem):
            pltpu.make_async_copy(invtab_hbm.at[cidx.at[j]], ib, sem).wait()
            pltpu.sync_copy(ib, inve_hbm.at[pl.ds((base + j) * CW, CW)])

        def pair(i, _):
            j0 = i * 2
            step(j0, ib0, s0)

            @pl.when(j0 + 2 < NCH)
            def _():
                pltpu.async_copy(invtab_hbm.at[cidx.at[j0 + 2]], ib0, s0)

            step(j0 + 1, ib1, s1)

            @pl.when(j0 + 3 < NCH)
            def _():
                pltpu.async_copy(invtab_hbm.at[cidx.at[j0 + 3]], ib1, s1)
            return 0
        lax.fori_loop(0, NCH // 2, pair, 0)
        if NCH % 2:
            step(NCH - 1, ib0, s0)

    return p0b


def _s2_build(NP, E, RN, HD):
    C = 80
    NCHT = E // C
    NROW = NP // _NS

    @functools.partial(
        pl.kernel,
        out_type=jax.ShapeDtypeStruct((_NC, NP, HD), jnp.float32),
        mesh=_mesh(),
        compiler_params=pltpu.CompilerParams(needs_layout_passes=False),
        scratch_types=[
            pltpu.VMEM((3, C), jnp.int32),
            pltpu.VMEM((3, C), jnp.int32),
            pltpu.VMEM((C, HD), jnp.float32),
            pltpu.VMEM((C, HD), jnp.float32),
            pltpu.VMEM((NROW, HD), jnp.float32),
            pltpu.VMEM((32,), jnp.int32),
            pltpu.SemaphoreType.DMA,
            pltpu.SemaphoreType.DMA,
        ],
    )
    def s2(h_hbm, meta_hbm, bnd_hbm, out_hbm,
           mb0, mb1, rb0, rb1, acc, bv, gs0, gs1):
        c = lax.axis_index("c")
        s = lax.axis_index("s")
        lo = s * NROW
        coff = c * RN
        zero = jnp.zeros((_LN,), jnp.float32)
        iota = lax.iota(jnp.int32, _LN)

        pltpu.sync_copy(bnd_hbm, bv)
        v0 = bv[pl.ds(0, _LN)]
        v1 = bv[pl.ds(_LN, _LN)]
        zi = jnp.zeros((_LN,), jnp.int32)
        b0 = jnp.sum(jnp.where(iota == s, v0, zi))
        b1 = (jnp.sum(jnp.where(iota == s + 1, v0, zi))
              + jnp.sum(jnp.where(iota + _LN == s + 1, v1, zi)))
        c0 = b0 // C
        c1 = (b1 + C - 1) // C
        n = c1 - c0

        def zrow(i, _):
            def zcol(g, _):
                acc[i, pl.ds(g * _LN, _LN)] = zero
                return 0
            lax.fori_loop(0, HD // _LN, zcol, 0)
            return 0
        lax.fori_loop(0, NROW, zrow, 0)

        def fetch(k, mb, rb, gs):
            pltpu.sync_copy(meta_hbm.at[k], mb)

            def addoff(g, _):
                mb[0, pl.ds(g * _LN, _LN)] = mb[0, pl.ds(g * _LN, _LN)] + coff
                return 0
            lax.fori_loop(0, C // _LN, addoff, 0)
            pltpu.async_copy(h_hbm.at[mb.at[0]], rb, gs)

        @pl.when(n > 0)
        def _():
            fetch(c0, mb0, rb0, gs0)

        @pl.when(n > 1)
        def _():
            fetch(c0 + 1, mb1, rb1, gs1)

        def process(mb, rb, gs):
            pltpu.make_async_copy(h_hbm.at[mb.at[0]], rb, gs).wait()
            zf = jnp.zeros((_LN,), jnp.float32)
            zi2 = jnp.zeros((_LN,), jnp.int32)

            dnums = lax.GatherDimensionNumbers(
                offset_dims=(), collapsed_slice_dims=(0,),
                start_index_map=(0,))

            def grp(g, _):
                dv = mb[1, pl.ds(g * _LN, _LN)] - lo
                m = (dv >= 0) & (dv < NROW)
                dvc = jnp.clip(dv, 0, NROW - 1)
                iv = plsc.bitcast(mb[2, pl.ds(g * _LN, _LN)], jnp.float32)
                scm = jnp.where(m, iv, zf)
                for e in range(_LN):
                    lane = iota == e
                    dve = jnp.sum(jnp.where(lane, dvc, zi2))
                    se = lax.gather(
                        scm, jnp.full((_LN, 1), e, jnp.int32), dnums,
                        slice_sizes=(1,),
                        mode=lax.GatherScatterMode.PROMISE_IN_BOUNDS)
                    r = g * _LN + e
                    for g2 in range(HD // _LN):
                        v = rb[r, pl.ds(g2 * _LN, _LN)]
                        acc[dve, pl.ds(g2 * _LN, _LN)] = (
                            acc[dve, pl.ds(g2 * _LN, _LN)] + v * se)
                return 0
            lax.fori_loop(0, C // _LN, grp, 0)

        def pair(i, _):
            j0 = c0 + i * 2
            process(mb0, rb0, gs0)

            @pl.when(j0 + 2 < c1)
            def _():
                fetch(j0 + 2, mb0, rb0, gs0)

            process(mb1, rb1, gs1)

            @pl.when(j0 + 3 < c1)
            def _():
                fetch(j0 + 3, mb1, rb1, gs1)
            return 0
        lax.fori_loop(0, n // 2, pair, 0)

        @pl.when(n % 2 == 1)
        def _():
            process(mb0, rb0, gs0)

        pltpu.sync_copy(acc, out_hbm.at[c, pl.ds(lo, NROW)])

    return s2


def _t1(X, W):
    N, D = X.shape
    R = W.shape[0]
    HD = D // 2
    BN = 2000

    def body(x_ref, w_ref, o_ref):
        acc = jnp.dot(x_ref[...], w_ref[0], preferred_element_type=jnp.float32)
        o_ref[0] = acc[:, :HD]
        o_ref[1] = acc[:, HD:]

    return pl.pallas_call(
        body,
        grid=(N // BN, R),
        in_specs=[
            pl.BlockSpec((BN, D), lambda i, r: (i, 0)),
            pl.BlockSpec((1, D, D), lambda i, r: (r, 0, 0)),
        ],
        out_specs=pl.BlockSpec((2, BN, HD),
                               lambda i, r, _n=N // BN: (0, r * _n + i, 0)),
        out_shape=jax.ShapeDtypeStruct((2, R * N, HD), jnp.float32),
    )(X, W)


def _t3(X, eacc, Wself):
    N, D = X.shape
    HD = D // 2
    BN = 2000

    def body(x_ref, e_ref, ws_ref, o_ref):
        e = jnp.concatenate([e_ref[0], e_ref[1]], axis=1)
        o_ref[...] = jnp.maximum(
            jnp.dot(x_ref[...], ws_ref[...],
                    preferred_element_type=jnp.float32) + e, 0.0)

    return pl.pallas_call(
        body,
        grid=(N // BN,),
        in_specs=[
            pl.BlockSpec((BN, D), lambda i: (i, 0)),
            pl.BlockSpec((2, BN, HD), lambda i: (0, i, 0)),
            pl.BlockSpec((D, D), lambda i: (0, 0)),
        ],
        out_specs=pl.BlockSpec((BN, D), lambda i: (i, 0)),
        out_shape=jax.ShapeDtypeStruct((N, D), jnp.float32),
    )(X, eacc, Wself)


def kernel(X, edge_index, edge_type, W0, Wself0, W1, Wself1):
    N, D = X.shape
    R = W0.shape[0]
    E = edge_type.shape[0]
    RN = R * N
    HD = D // 2
    NP = ((N + 1023) // 1024) * 1024
    NH = NP // 2

    src = edge_index[0].astype(jnp.int32)
    dst = edge_index[1].astype(jnp.int32)
    et = edge_type.astype(jnp.int32)

    order = jnp.argsort(dst)
    dsts = dst[order]
    combs = et[order] * N + dsts
    gidxs = (et * N + src)[order]

    NROW = NP // _NS
    bnd = jnp.searchsorted(
        dsts, jnp.arange(_NS + 1) * NROW).astype(jnp.int32)
    bnd32 = jnp.concatenate([bnd, jnp.full((32 - _NS - 1,), E, jnp.int32)])

    invtab = _p0a_build(E, RN)(combs)
    invs = _p0b_build(E, RN)(combs.reshape(_NW, E // (_NW * 40), 40), invtab)

    meta = jnp.stack(
        [gidxs, dsts, jax.lax.bitcast_convert_type(invs, jnp.int32)]
    ).reshape(3, E // 80, 80).transpose(1, 0, 2)

    s2 = _s2_build(NP, E, RN, HD)

    def layer(h, W, Wself):
        hrel = _t1(h, W).reshape(2 * RN, HD)
        eacc = s2(hrel, meta, bnd32)[:, :N]
        return _t3(h, eacc, Wself)

    h = layer(X, W0, Wself0)
    return layer(h, W1, Wself1)
